# baseline (speedup 1.0000x reference)
import functools

import jax
import jax.numpy as jnp
from jax import lax
from jax.experimental import pallas as pl
from jax.experimental.pallas import tpu as pltpu
from jax.experimental.pallas import tpu_sc as plsc

NUM_CLASSES = 100000
FEATURE_DIM = 128
BATCH = 16384

_NC = 2
_NS = 16
_NW = _NC * _NS
_BPW0 = 528
_BPW1 = 496


def _gather_kernel(centers_hbm, idx_hbm, out_hbm, idx_v, rows_v, sem):
    c = lax.axis_index("c")
    s = lax.axis_index("s")

    def work(base, n):
        pltpu.sync_copy(idx_hbm.at[pl.ds(base, n)], idx_v.at[pl.ds(0, n)])
        pltpu.async_copy(
            centers_hbm.at[idx_v.at[pl.ds(0, n)]], rows_v.at[pl.ds(0, n)], sem
        ).wait()
        pltpu.sync_copy(rows_v.at[pl.ds(0, n)], out_hbm.at[pl.ds(base, n)])

    @pl.when(c == 0)
    def _():
        work(s * _BPW0, _BPW0)

    @pl.when(c == 1)
    def _():
        work(_NS * _BPW0 + s * _BPW1, _BPW1)


@jax.jit
def kernel(labels, centers):
    idx = labels.astype(jnp.int32)
    mesh = plsc.VectorSubcoreMesh(core_axis_name="c", subcore_axis_name="s")
    return pl.kernel(
        _gather_kernel,
        mesh=mesh,
        out_type=jax.ShapeDtypeStruct((BATCH, FEATURE_DIM), jnp.float32),
        scratch_types=[
            pltpu.VMEM((_BPW0,), jnp.int32),
            pltpu.VMEM((_BPW0, FEATURE_DIM), jnp.float32),
            pltpu.SemaphoreType.DMA,
        ],
    )(centers, idx)


# speedup vs baseline: 1.0129x; 1.0129x over previous
import functools

import jax
import jax.numpy as jnp
from jax import lax
from jax.experimental import pallas as pl
from jax.experimental.pallas import tpu as pltpu
from jax.experimental.pallas import tpu_sc as plsc

NUM_CLASSES = 100000
FEATURE_DIM = 128
BATCH = 16384

_NC = 2
_NS = 16
_NW = _NC * _NS
_BPW0 = 496
_BPW1 = 528


def _gather_kernel(centers_hbm, idx_hbm, out_hbm, idx_v, rows_v, sem):
    c = lax.axis_index("c")
    s = lax.axis_index("s")

    def work(base, n):
        pltpu.sync_copy(idx_hbm.at[pl.ds(base, n)], idx_v.at[pl.ds(0, n)])
        pltpu.async_copy(
            centers_hbm.at[idx_v.at[pl.ds(0, n)]], rows_v.at[pl.ds(0, n)], sem
        ).wait()
        pltpu.sync_copy(rows_v.at[pl.ds(0, n)], out_hbm.at[pl.ds(base, n)])

    @pl.when(c == 0)
    def _():
        work(s * _BPW0, _BPW0)

    @pl.when(c == 1)
    def _():
        work(_NS * _BPW0 + s * _BPW1, _BPW1)


@jax.jit
def kernel(labels, centers):
    idx = labels.astype(jnp.int32)
    mesh = plsc.VectorSubcoreMesh(core_axis_name="c", subcore_axis_name="s")
    return pl.kernel(
        _gather_kernel,
        mesh=mesh,
        out_type=jax.ShapeDtypeStruct((BATCH, FEATURE_DIM), jnp.float32),
        scratch_types=[
            pltpu.VMEM((_BPW1,), jnp.int32),
            pltpu.VMEM((_BPW1, FEATURE_DIM), jnp.float32),
            pltpu.SemaphoreType.DMA,
        ],
    )(centers, idx)


# speedup vs baseline: 1.0190x; 1.0061x over previous
import jax
import jax.numpy as jnp
from jax import lax
from jax.experimental import pallas as pl
from jax.experimental.pallas import tpu as pltpu
from jax.experimental.pallas import tpu_sc as plsc

NUM_CLASSES = 100000
FEATURE_DIM = 128
BATCH = 16384

_NC = 2
_NS = 16
_NW = _NC * _NS
_BPW = BATCH // _NW


def _gather_kernel(centers_hbm, idx_hbm, out_hbm, idx_v, rows_v, sem):
    wid = lax.axis_index("s") * _NC + lax.axis_index("c")
    base = wid * _BPW
    pltpu.sync_copy(idx_hbm.at[pl.ds(base, _BPW)], idx_v)
    pltpu.async_copy(centers_hbm.at[idx_v], rows_v, sem).wait()
    pltpu.sync_copy(rows_v, out_hbm.at[pl.ds(base, _BPW)])


@jax.jit
def kernel(labels, centers):
    idx = labels.astype(jnp.int32)
    mesh = plsc.VectorSubcoreMesh(core_axis_name="c", subcore_axis_name="s")
    return pl.kernel(
        _gather_kernel,
        mesh=mesh,
        out_type=jax.ShapeDtypeStruct((BATCH, FEATURE_DIM), jnp.float32),
        scratch_types=[
            pltpu.VMEM((_BPW,), jnp.int32),
            pltpu.VMEM((_BPW, FEATURE_DIM), jnp.float32),
            pltpu.SemaphoreType.DMA,
        ],
    )(centers, idx)
